# Initial kernel scaffold; baseline (speedup 1.0000x reference)
import jax
import jax.numpy as jnp
from jax.experimental import pallas as pl


def kernel(x, edge_index, batch, edge_attr, W1, b1, W2, b2, W3, b3, W4):
    raise NotImplementedError("write your pallas kernel here")


# speedup vs baseline: 23.8977x; 23.8977x over previous
import functools

import jax
import jax.numpy as jnp
from jax import lax
from jax.experimental import pallas as pl
from jax.experimental.pallas import tpu as pltpu
from jax.experimental.pallas import tpu_sc as plsc

N = 10000
NP = 10112
E = 320000
NSUB = 16
NT = NP // NSUB
ET = 20480
EP = NSUB * ET
CH = 128
NCHUNK = ET // CH

_SC_MESH = dict(core_axis_name="c", subcore_axis_name="s")


def _prep_body(src_ref, dst_ref, dstr_ref, sdeg_ref):
    s = src_ref[...]
    d = dst_ref[...]
    lane = lax.broadcasted_iota(jnp.int32, s.shape, 1) % 16
    bad = (s == d) | (d < 0)
    dummy = N + lane
    dstr_ref[...] = jnp.where(bad, dummy, d)
    sdeg_ref[...] = jnp.where(bad, dummy, s)


def _prep_edges(src_p, dst_p):
    s2 = src_p.reshape(EP // 128, 128)
    d2 = dst_p.reshape(EP // 128, 128)
    return pl.pallas_call(
        _prep_body,
        out_shape=(jax.ShapeDtypeStruct(s2.shape, jnp.int32),
                   jax.ShapeDtypeStruct(s2.shape, jnp.int32)),
    )(s2, d2)


def _dinv_body(deg_ref, dinv_ref):
    deg = deg_ref[...]
    dinv_ref[...] = jnp.where(deg > 0, lax.rsqrt(jnp.maximum(deg, 1e-12)), 0.0)


def _dinv(deg16):
    return pl.pallas_call(
        _dinv_body,
        out_shape=jax.ShapeDtypeStruct((NP, 16), jnp.float32),
    )(deg16)


def _expand_body(h_ref, w_ref, b_ref, c_ref, *, silu_in, in_halves):
    h = h_ref[...]
    if in_halves:
        h = jnp.concatenate([h[0], h[1]], axis=1)
    if silu_in:
        h = h * jax.nn.sigmoid(h)
    out = jnp.dot(h, w_ref[...][0], preferred_element_type=jnp.float32)
    k = pl.program_id(0)
    out = out + b_ref[...][0] * (k == 0).astype(jnp.float32)
    c_ref[...] = out[None, None]


def _expand(h, Wr, br, K, silu_in, in_halves):
    IC = Wr.shape[1]
    if in_halves:
        h_spec = pl.BlockSpec((2, NP, 16), lambda k, hf: (0, 0, 0))
    else:
        h_spec = pl.BlockSpec((NP, IC), lambda k, hf: (0, 0))
    return pl.pallas_call(
        functools.partial(_expand_body, silu_in=silu_in, in_halves=in_halves),
        grid=(K, 2),
        in_specs=[
            h_spec,
            pl.BlockSpec((1, IC, 16), lambda k, hf: (2 * k + hf, 0, 0)),
            pl.BlockSpec((1, 1, 16), lambda k, hf: (hf, 0, 0)),
        ],
        out_specs=pl.BlockSpec((1, 1, NP, 16), lambda k, hf: (hf, k, 0, 0)),
        out_shape=jax.ShapeDtypeStruct((2, K, NP, 16), jnp.float32),
    )(h, Wr, br)


def _final_body(h_ref, w_ref, o_ref):
    h = jnp.concatenate([h_ref[...][0], h_ref[...][1]], axis=1)
    h = h * jax.nn.sigmoid(h)
    o_ref[...] = jnp.dot(h, w_ref[...], preferred_element_type=jnp.float32)


def _final(h, W4):
    return pl.pallas_call(
        _final_body,
        out_shape=jax.ShapeDtypeStruct((NP, 128), jnp.float32),
    )(h, W4)


def _deg_kernel_body(sdeg_hbm, ones_hbm, zeros_hbm, deg_out,
                     tdeg_sh, idx_v, ones_v, sem):
    cid = lax.axis_index("c")
    sid = lax.axis_index("s")

    @pl.when(cid == 0)
    def _():
        n0 = sid * NT
        pltpu.sync_copy(zeros_hbm.at[pl.ds(n0, NT)], tdeg_sh.at[pl.ds(n0, NT)])
        pltpu.sync_copy(ones_hbm, ones_v)
        pltpu.sync_copy(sdeg_hbm.at[sid], idx_v)
        plsc.subcore_barrier()

        @pl.loop(0, NCHUNK)
        def _(j):
            pltpu.async_copy(ones_v, tdeg_sh.at[idx_v.at[j]], sem,
                             add=True).wait()

        plsc.subcore_barrier()
        pltpu.sync_copy(tdeg_sh.at[pl.ds(n0, NT)], deg_out.at[pl.ds(n0, NT)])


def _degree(sdeg4, ones_hbm, zeros_hbm):
    return pl.kernel(
        _deg_kernel_body,
        out_type=jax.ShapeDtypeStruct((NP, 16), jnp.float32),
        mesh=plsc.VectorSubcoreMesh(**_SC_MESH),
        compiler_params=pltpu.CompilerParams(use_tc_tiling_on_sc=False),
        scratch_types=[
            pltpu.VMEM_SHARED((NP, 16), jnp.float32),
            pltpu.VMEM((NCHUNK, CH), jnp.int32),
            pltpu.VMEM((CH, 16), jnp.float32),
            pltpu.SemaphoreType.DMA,
        ],
    )(sdeg4, ones_hbm, zeros_hbm)


def _chain_body(c_hbm, dinv_hbm, src_hbm, dst_hbm, zn_hbm, h_out,
                sP, sQ, sZ, sS,
                srcv, dstv, rb0, rb1,
                dbuf, cbuf, sbuf, qbuf,
                g0, g1, s0, s1, *, K):
    cid = lax.axis_index("c")
    sid = lax.axis_index("s")
    n0 = sid * NT
    nsl = pl.ds(n0, NT)

    pltpu.sync_copy(src_hbm.at[sid], srcv)
    pltpu.sync_copy(dst_hbm.at[sid], dstv)
    pltpu.sync_copy(dinv_hbm.at[nsl], dbuf)

    pltpu.sync_copy(c_hbm.at[cid, K - 1, nsl], cbuf)

    @pl.loop(0, NT)
    def _(i):
        sbuf[i] = cbuf[i] * dbuf[i]

    pltpu.sync_copy(cbuf, sP.at[nsl])
    pltpu.sync_copy(sbuf, sZ.at[nsl])
    pltpu.sync_copy(zn_hbm, sQ.at[nsl])
    pltpu.sync_copy(zn_hbm, sS.at[nsl])
    plsc.subcore_barrier()

    def scatter_phase():
        @pl.loop(0, NCHUNK // 2)
        def _(j):
            c0 = 2 * j
            c1 = 2 * j + 1
            pltpu.async_copy(sZ.at[srcv.at[c0]], rb0, g0).wait()
            cp0 = pltpu.async_copy(rb0, sS.at[dstv.at[c0]], s0, add=True)
            pltpu.async_copy(sZ.at[srcv.at[c1]], rb1, g1).wait()
            cp0.wait()
            pltpu.async_copy(rb1, sS.at[dstv.at[c1]], s1, add=True).wait()

    def ew_step(k, qref):
        pltpu.sync_copy(c_hbm.at[cid, k, nsl], cbuf)
        pltpu.sync_copy(sS.at[nsl], sbuf)
        pltpu.sync_copy(qref.at[nsl], qbuf)

        @pl.loop(0, NT)
        def _(i):
            d = dbuf[i]
            bnew = cbuf[i] - 2.0 * d * sbuf[i] - qbuf[i]
            cbuf[i] = bnew
            sbuf[i] = d * bnew

        pltpu.sync_copy(cbuf, qref.at[nsl])
        pltpu.sync_copy(sbuf, sZ.at[nsl])
        pltpu.sync_copy(zn_hbm, sS.at[nsl])

    @pl.loop(0, (K - 2) // 2)
    def _(t):
        k_even = (K - 2) - 2 * t
        scatter_phase()
        plsc.subcore_barrier()
        ew_step(k_even, sQ)
        plsc.subcore_barrier()
        scatter_phase()
        plsc.subcore_barrier()
        ew_step(k_even - 1, sP)
        plsc.subcore_barrier()

    scatter_phase()
    plsc.subcore_barrier()
    pltpu.sync_copy(c_hbm.at[cid, 0, nsl], cbuf)
    pltpu.sync_copy(sS.at[nsl], sbuf)
    pltpu.sync_copy(sQ.at[nsl], qbuf)

    @pl.loop(0, NT)
    def _(i):
        cbuf[i] = cbuf[i] - dbuf[i] * sbuf[i] - qbuf[i]

    pltpu.sync_copy(cbuf, h_out.at[cid, nsl])


def _chain(C, dinv16, src4, dst4, zn, K):
    return pl.kernel(
        functools.partial(_chain_body, K=K),
        out_type=jax.ShapeDtypeStruct((2, NP, 16), jnp.float32),
        mesh=plsc.VectorSubcoreMesh(**_SC_MESH),
        compiler_params=pltpu.CompilerParams(use_tc_tiling_on_sc=False),
        scratch_types=[
            pltpu.VMEM_SHARED((NP, 16), jnp.float32),
            pltpu.VMEM_SHARED((NP, 16), jnp.float32),
            pltpu.VMEM_SHARED((NP, 16), jnp.float32),
            pltpu.VMEM_SHARED((NP, 16), jnp.float32),
            pltpu.VMEM((NCHUNK, CH), jnp.int32),
            pltpu.VMEM((NCHUNK, CH), jnp.int32),
            pltpu.VMEM((CH, 16), jnp.float32),
            pltpu.VMEM((CH, 16), jnp.float32),
            pltpu.VMEM((NT, 16), jnp.float32),
            pltpu.VMEM((NT, 16), jnp.float32),
            pltpu.VMEM((NT, 16), jnp.float32),
            pltpu.VMEM((NT, 16), jnp.float32),
            pltpu.SemaphoreType.DMA,
            pltpu.SemaphoreType.DMA,
            pltpu.SemaphoreType.DMA,
            pltpu.SemaphoreType.DMA,
        ],
    )(C, dinv16, src4, dst4, zn)


def kernel(x, edge_index, batch, edge_attr, W1, b1, W2, b2, W3, b3, W4):
    del batch, edge_attr
    src = edge_index[0].astype(jnp.int32)
    dst = edge_index[1].astype(jnp.int32)
    pad = EP - E
    src_p = jnp.concatenate([src, jnp.zeros((pad,), jnp.int32)])
    dst_p = jnp.concatenate([dst, jnp.full((pad,), -1, jnp.int32)])

    dst_r, sdeg = _prep_edges(src_p, dst_p)
    src4 = src_p.reshape(NSUB, NCHUNK, CH)
    dst4 = dst_r.reshape(NSUB, NCHUNK, CH)
    sdeg4 = sdeg.reshape(NSUB, NCHUNK, CH)

    deg16 = _degree(sdeg4, jnp.ones((CH, 16), jnp.float32),
                    jnp.zeros((NP, 16), jnp.float32))
    dinv16 = _dinv(deg16)

    def wshape(W):
        K, IC, _ = W.shape
        return W.reshape(K, IC, 2, 16).transpose(0, 2, 1, 3).reshape(2 * K, IC, 16)

    def bshape(b):
        return b.reshape(2, 1, 16)

    x_pad = jnp.concatenate([x, jnp.zeros((NP - N, 128), jnp.float32)])

    C1 = _expand(x_pad, wshape(W1), bshape(b1), 24, silu_in=False, in_halves=False)
    zn = jnp.zeros((NT, 16), jnp.float32)
    h1 = _chain(C1, dinv16, src4, dst4, zn, 24)

    C2 = _expand(h1, wshape(W2), bshape(b2), 12, silu_in=True, in_halves=True)
    h2 = _chain(C2, dinv16, src4, dst4, zn, 12)

    C3 = _expand(h2, wshape(W3), bshape(b3), 10, silu_in=True, in_halves=True)
    h3 = _chain(C3, dinv16, src4, dst4, zn, 10)

    out = _final(h3, W4[0])
    return out[:N]


# speedup vs baseline: 28.5903x; 1.1964x over previous
import functools

import jax
import jax.numpy as jnp
from jax import lax
from jax.experimental import pallas as pl
from jax.experimental.pallas import tpu as pltpu
from jax.experimental.pallas import tpu_sc as plsc

N = 10000
NP = 10112
E = 320000
NSUB = 16
NT = NP // NSUB
ET = 20480
EP = NSUB * ET
CH = 640
NCHUNK = ET // CH

_SC_MESH = dict(core_axis_name="c", subcore_axis_name="s")


def _prep_body(src_ref, dst_ref, dstr_ref, sdeg_ref):
    s = src_ref[...]
    d = dst_ref[...]
    lane = lax.broadcasted_iota(jnp.int32, s.shape, 1) % 16
    bad = (s == d) | (d < 0)
    dummy = N + lane
    dstr_ref[...] = jnp.where(bad, dummy, d)
    sdeg_ref[...] = jnp.where(bad, dummy, s)


def _prep_edges(src_p, dst_p):
    s2 = src_p.reshape(EP // 128, 128)
    d2 = dst_p.reshape(EP // 128, 128)
    return pl.pallas_call(
        _prep_body,
        out_shape=(jax.ShapeDtypeStruct(s2.shape, jnp.int32),
                   jax.ShapeDtypeStruct(s2.shape, jnp.int32)),
    )(s2, d2)


def _dinv_body(deg_ref, dinv_ref):
    deg = deg_ref[...]
    dinv_ref[...] = jnp.where(deg > 0, lax.rsqrt(jnp.maximum(deg, 1e-12)), 0.0)


def _dinv(deg16):
    return pl.pallas_call(
        _dinv_body,
        out_shape=jax.ShapeDtypeStruct((NP, 16), jnp.float32),
    )(deg16)


def _expand_body(h_ref, w_ref, b_ref, c_ref, *, silu_in, in_halves):
    h = h_ref[...]
    if in_halves:
        h = jnp.concatenate([h[0], h[1]], axis=1)
    if silu_in:
        h = h * jax.nn.sigmoid(h)
    out = jnp.dot(h, w_ref[...][0], preferred_element_type=jnp.float32)
    k = pl.program_id(0)
    out = out + b_ref[...][0] * (k == 0).astype(jnp.float32)
    c_ref[...] = out[None, None]


def _expand(h, Wr, br, K, silu_in, in_halves):
    IC = Wr.shape[1]
    if in_halves:
        h_spec = pl.BlockSpec((2, NP, 16), lambda k, hf: (0, 0, 0))
    else:
        h_spec = pl.BlockSpec((NP, IC), lambda k, hf: (0, 0))
    return pl.pallas_call(
        functools.partial(_expand_body, silu_in=silu_in, in_halves=in_halves),
        grid=(K, 2),
        in_specs=[
            h_spec,
            pl.BlockSpec((1, IC, 16), lambda k, hf: (2 * k + hf, 0, 0)),
            pl.BlockSpec((1, 1, 16), lambda k, hf: (hf, 0, 0)),
        ],
        out_specs=pl.BlockSpec((1, 1, NP, 16), lambda k, hf: (hf, k, 0, 0)),
        out_shape=jax.ShapeDtypeStruct((2, K, NP, 16), jnp.float32),
    )(h, Wr, br)


def _final_body(h_ref, w_ref, o_ref):
    h = jnp.concatenate([h_ref[...][0], h_ref[...][1]], axis=1)
    h = h * jax.nn.sigmoid(h)
    o_ref[...] = jnp.dot(h, w_ref[...], preferred_element_type=jnp.float32)


def _final(h, W4):
    return pl.pallas_call(
        _final_body,
        out_shape=jax.ShapeDtypeStruct((NP, 128), jnp.float32),
    )(h, W4)


def _deg_kernel_body(sdeg_hbm, ones_hbm, zeros_hbm, deg_out,
                     tdeg_sh, idx_v, ones_v, sem):
    cid = lax.axis_index("c")
    sid = lax.axis_index("s")

    @pl.when(cid == 0)
    def _():
        n0 = sid * NT
        pltpu.sync_copy(zeros_hbm.at[pl.ds(n0, NT)], tdeg_sh.at[pl.ds(n0, NT)])
        pltpu.sync_copy(ones_hbm, ones_v)
        pltpu.sync_copy(sdeg_hbm.at[sid], idx_v)
        plsc.subcore_barrier()

        @pl.loop(0, NCHUNK)
        def _(j):
            pltpu.async_copy(ones_v, tdeg_sh.at[idx_v.at[j]], sem,
                             add=True).wait()

        plsc.subcore_barrier()
        pltpu.sync_copy(tdeg_sh.at[pl.ds(n0, NT)], deg_out.at[pl.ds(n0, NT)])


def _degree(sdeg4, ones_hbm, zeros_hbm):
    return pl.kernel(
        _deg_kernel_body,
        out_type=jax.ShapeDtypeStruct((NP, 16), jnp.float32),
        mesh=plsc.VectorSubcoreMesh(**_SC_MESH),
        compiler_params=pltpu.CompilerParams(use_tc_tiling_on_sc=False),
        scratch_types=[
            pltpu.VMEM_SHARED((NP, 16), jnp.float32),
            pltpu.VMEM((NCHUNK, CH), jnp.int32),
            pltpu.VMEM((CH, 16), jnp.float32),
            pltpu.SemaphoreType.DMA,
        ],
    )(sdeg4, ones_hbm, zeros_hbm)


def _chain_body(c_hbm, dinv_hbm, src_hbm, dst_hbm, zn_hbm, h_out,
                sP, sQ, sZ, sS,
                srcv, dstv, rb0, rb1,
                dbuf, qbuf,
                g0, g1, s0, s1, *, K):
    cid = lax.axis_index("c")
    sid = lax.axis_index("s")
    n0 = sid * NT
    nsl = pl.ds(n0, NT)
    tsl = pl.ds(0, NT)
    cbuf = rb0.at[tsl]
    sbuf = rb1.at[tsl]

    pltpu.sync_copy(src_hbm.at[sid], srcv)
    pltpu.sync_copy(dst_hbm.at[sid], dstv)
    pltpu.sync_copy(dinv_hbm.at[nsl], dbuf)

    pltpu.sync_copy(c_hbm.at[cid, K - 1, nsl], cbuf)

    @pl.loop(0, NT)
    def _(i):
        sbuf[i] = cbuf[i] * dbuf[i]

    pltpu.sync_copy(cbuf, sP.at[nsl])
    pltpu.sync_copy(sbuf, sZ.at[nsl])
    pltpu.sync_copy(zn_hbm, sQ.at[nsl])
    pltpu.sync_copy(zn_hbm, sS.at[nsl])
    plsc.subcore_barrier()

    def scatter_phase():
        T = NCHUNK // 2
        pltpu.async_copy(sZ.at[srcv.at[0]], rb0, g0)

        @pl.loop(0, T)
        def _(t):
            c0 = 2 * t
            pltpu.make_async_copy(sZ.at[srcv.at[0]], rb0, g0).wait()

            @pl.when(t > 0)
            def _():
                pltpu.make_async_copy(rb1, sS.at[dstv.at[0]], s1).wait()

            pltpu.async_copy(sZ.at[srcv.at[c0 + 1]], rb1, g1)
            pltpu.async_copy(rb0, sS.at[dstv.at[c0]], s0, add=True)
            pltpu.make_async_copy(sZ.at[srcv.at[0]], rb1, g1).wait()
            pltpu.make_async_copy(rb0, sS.at[dstv.at[0]], s0).wait()

            @pl.when(t + 1 < T)
            def _():
                pltpu.async_copy(sZ.at[srcv.at[c0 + 2]], rb0, g0)

            pltpu.async_copy(rb1, sS.at[dstv.at[c0 + 1]], s1, add=True)

        pltpu.make_async_copy(rb1, sS.at[dstv.at[0]], s1).wait()

    def ew_step(k, qref):
        pltpu.sync_copy(c_hbm.at[cid, k, nsl], cbuf)
        pltpu.sync_copy(sS.at[nsl], sbuf)
        pltpu.sync_copy(qref.at[nsl], qbuf)

        @pl.loop(0, NT)
        def _(i):
            d = dbuf[i]
            bnew = cbuf[i] - 2.0 * d * sbuf[i] - qbuf[i]
            cbuf[i] = bnew
            sbuf[i] = d * bnew

        pltpu.sync_copy(cbuf, qref.at[nsl])
        pltpu.sync_copy(sbuf, sZ.at[nsl])
        pltpu.sync_copy(zn_hbm, sS.at[nsl])

    @pl.loop(0, (K - 2) // 2)
    def _(t):
        k_even = (K - 2) - 2 * t
        scatter_phase()
        plsc.subcore_barrier()
        ew_step(k_even, sQ)
        plsc.subcore_barrier()
        scatter_phase()
        plsc.subcore_barrier()
        ew_step(k_even - 1, sP)
        plsc.subcore_barrier()

    scatter_phase()
    plsc.subcore_barrier()
    pltpu.sync_copy(c_hbm.at[cid, 0, nsl], cbuf)
    pltpu.sync_copy(sS.at[nsl], sbuf)
    pltpu.sync_copy(sQ.at[nsl], qbuf)

    @pl.loop(0, NT)
    def _(i):
        cbuf[i] = cbuf[i] - dbuf[i] * sbuf[i] - qbuf[i]

    pltpu.sync_copy(cbuf, h_out.at[cid, nsl])


def _chain(C, dinv16, src4, dst4, zn, K):
    return pl.kernel(
        functools.partial(_chain_body, K=K),
        out_type=jax.ShapeDtypeStruct((2, NP, 16), jnp.float32),
        mesh=plsc.VectorSubcoreMesh(**_SC_MESH),
        compiler_params=pltpu.CompilerParams(use_tc_tiling_on_sc=False),
        scratch_types=[
            pltpu.VMEM_SHARED((NP, 16), jnp.float32),
            pltpu.VMEM_SHARED((NP, 16), jnp.float32),
            pltpu.VMEM_SHARED((NP, 16), jnp.float32),
            pltpu.VMEM_SHARED((NP, 16), jnp.float32),
            pltpu.VMEM((NCHUNK, CH), jnp.int32),
            pltpu.VMEM((NCHUNK, CH), jnp.int32),
            pltpu.VMEM((CH, 16), jnp.float32),
            pltpu.VMEM((CH, 16), jnp.float32),
            pltpu.VMEM((NT, 16), jnp.float32),
            pltpu.VMEM((NT, 16), jnp.float32),
            pltpu.SemaphoreType.DMA,
            pltpu.SemaphoreType.DMA,
            pltpu.SemaphoreType.DMA,
            pltpu.SemaphoreType.DMA,
        ],
    )(C, dinv16, src4, dst4, zn)


def kernel(x, edge_index, batch, edge_attr, W1, b1, W2, b2, W3, b3, W4):
    del batch, edge_attr
    src = edge_index[0].astype(jnp.int32)
    dst = edge_index[1].astype(jnp.int32)
    pad = EP - E
    src_p = jnp.concatenate([src, jnp.zeros((pad,), jnp.int32)])
    dst_p = jnp.concatenate([dst, jnp.full((pad,), -1, jnp.int32)])

    dst_r, sdeg = _prep_edges(src_p, dst_p)
    src4 = src_p.reshape(NSUB, NCHUNK, CH)
    dst4 = dst_r.reshape(NSUB, NCHUNK, CH)
    sdeg4 = sdeg.reshape(NSUB, NCHUNK, CH)

    deg16 = _degree(sdeg4, jnp.ones((CH, 16), jnp.float32),
                    jnp.zeros((NP, 16), jnp.float32))
    dinv16 = _dinv(deg16)

    def wshape(W):
        K, IC, _ = W.shape
        return W.reshape(K, IC, 2, 16).transpose(0, 2, 1, 3).reshape(2 * K, IC, 16)

    def bshape(b):
        return b.reshape(2, 1, 16)

    x_pad = jnp.concatenate([x, jnp.zeros((NP - N, 128), jnp.float32)])

    C1 = _expand(x_pad, wshape(W1), bshape(b1), 24, silu_in=False, in_halves=False)
    zn = jnp.zeros((NT, 16), jnp.float32)
    h1 = _chain(C1, dinv16, src4, dst4, zn, 24)

    C2 = _expand(h1, wshape(W2), bshape(b2), 12, silu_in=True, in_halves=True)
    h2 = _chain(C2, dinv16, src4, dst4, zn, 12)

    C3 = _expand(h2, wshape(W3), bshape(b3), 10, silu_in=True, in_halves=True)
    h3 = _chain(C3, dinv16, src4, dst4, zn, 10)

    out = _final(h3, W4[0])
    return out[:N]
